# baseline (speedup 1.0000x reference)
import functools

import jax
import jax.numpy as jnp
from jax import lax
from jax.experimental import pallas as pl
from jax.experimental.pallas import tpu as pltpu
from jax.experimental.pallas import tpu_sc as plsc

D = 768
H = 2048
E = 64
S = 2048
CHUNK = 128
SLOT_PAD = 2816
SH_CHUNKS = (S + E * 8 + CHUNK - 1) // CHUNK


def _router_body(x_ref, rw_ref, logits_ref, eid_ref):
    lg = jnp.dot(x_ref[...], rw_ref[...], preferred_element_type=jnp.float32)
    logits_ref[...] = lg
    maxv = jnp.max(lg, axis=1, keepdims=True)
    col = lax.broadcasted_iota(jnp.int32, lg.shape, 1)
    ids = jnp.min(jnp.where(lg == maxv, col, E), axis=1, keepdims=True)
    eid_ref[...] = jnp.broadcast_to(ids, lg.shape).astype(jnp.int32)


_router = pl.pallas_call(
    _router_body,
    out_shape=(
        jax.ShapeDtypeStruct((S, E), jnp.float32),
        jax.ShapeDtypeStruct((S, E), jnp.int32),
    ),
)


_NC = 2
_NSC = 16
_NW = _NC * _NSC


@functools.lru_cache(maxsize=None)
def _make_row_gather(n_rows):
    b_per_w = n_rows // _NW
    mesh = plsc.VectorSubcoreMesh(
        core_axis_name="c", subcore_axis_name="s",
        num_cores=_NC, num_subcores=_NSC)

    @functools.partial(
        pl.kernel,
        mesh=mesh,
        out_type=jax.ShapeDtypeStruct((n_rows, D), jnp.float32),
        scratch_types=[
            pltpu.VMEM((b_per_w,), jnp.int32),
            pltpu.VMEM((b_per_w, D), jnp.float32),
            pltpu.SemaphoreType.DMA,
        ],
    )
    def k(table_hbm, idx_hbm, out_hbm, idx_v, rows_v, sem):
        wid = lax.axis_index("s") * _NC + lax.axis_index("c")
        base = wid * b_per_w
        pltpu.sync_copy(idx_hbm.at[pl.ds(base, b_per_w)], idx_v)
        pltpu.async_copy(table_hbm.at[idx_v], rows_v, sem).wait()
        pltpu.sync_copy(rows_v, out_hbm.at[pl.ds(base, b_per_w)])

    return k


def _silu(v):
    return v * jax.nn.sigmoid(v)


def _moe_body(meta_ref, x_ref, w1_ref, w2_ref, cp_ref, out_ref):
    g = pl.program_id(0)
    start = pl.multiple_of(meta_ref[g], 8)
    count = meta_ref[E + 1 + g]
    nch = (count + CHUNK - 1) // CHUNK

    def body(i, carry):
        r0 = start + i * CHUNK
        rows = x_ref[pl.ds(r0, CHUNK), :]
        a = jnp.dot(rows, w1_ref[0], preferred_element_type=jnp.float32)
        b = jnp.dot(rows, w2_ref[0], preferred_element_type=jnp.float32)
        hh = _silu(a) * b
        out_ref[pl.ds(r0, CHUNK), :] = jnp.dot(
            hh, cp_ref[0], preferred_element_type=jnp.float32)
        return carry

    lax.fori_loop(0, nch, body, 0)


_moe_grid = pltpu.PrefetchScalarGridSpec(
    num_scalar_prefetch=1,
    grid=(E,),
    in_specs=[
        pl.BlockSpec((SLOT_PAD, D), lambda g, offs: (0, 0)),
        pl.BlockSpec((1, D, H), lambda g, offs: (g, 0, 0)),
        pl.BlockSpec((1, D, H), lambda g, offs: (g, 0, 0)),
        pl.BlockSpec((1, H, D), lambda g, offs: (g, 0, 0)),
    ],
    out_specs=pl.BlockSpec((SLOT_PAD, D), lambda g, offs: (0, 0)),
)

_moe = pl.pallas_call(
    _moe_body,
    grid_spec=_moe_grid,
    out_shape=jax.ShapeDtypeStruct((SLOT_PAD, D), jnp.float32),
    compiler_params=pltpu.CompilerParams(
        dimension_semantics=("arbitrary",),
    ),
)


def _shared_body(x_ref, sw1_ref, sw2_ref, sc_ref, out_ref):
    def body(i, carry):
        r0 = i * CHUNK
        rows = x_ref[pl.ds(r0, CHUNK), :]
        a = jnp.dot(rows, sw1_ref[...], preferred_element_type=jnp.float32)
        b = jnp.dot(rows, sw2_ref[...], preferred_element_type=jnp.float32)
        hh = _silu(a) * b
        out_ref[pl.ds(r0, CHUNK), :] = jnp.dot(
            hh, sc_ref[...], preferred_element_type=jnp.float32)
        return carry

    lax.fori_loop(0, S // CHUNK, body, 0)


_shared = pl.pallas_call(
    _shared_body,
    out_shape=jax.ShapeDtypeStruct((S, D), jnp.float32),
)


def kernel(x, router_w, w1, w2, c_proj, s_w1, s_w2, s_c):
    b, s, d = x.shape
    x_flat = x.reshape(s, d)

    logits2d, eid2d = _router(x_flat, router_w)
    eid = eid2d[:, 0]

    perm = jnp.argsort(eid).astype(jnp.int32)
    counts = jnp.zeros((E,), jnp.int32).at[eid].add(1)
    offsets = jnp.concatenate(
        [jnp.zeros((1,), jnp.int32), jnp.cumsum(counts).astype(jnp.int32)])
    pc = (counts + 7) // 8 * 8
    aoff = jnp.concatenate(
        [jnp.zeros((1,), jnp.int32), jnp.cumsum(pc).astype(jnp.int32)])
    es = eid[perm]
    slot = aoff[es] + jnp.arange(S, dtype=jnp.int32) - offsets[es]
    src = jnp.zeros((SLOT_PAD,), jnp.int32).at[slot].set(perm)
    pos = jnp.zeros((S,), jnp.int32).at[perm].set(slot)
    meta = jnp.concatenate([aoff, counts])

    shared_flat = _shared(x_flat, s_w1[0], s_w2[0], s_c[0])
    x_sorted = _make_row_gather(SLOT_PAD)(x_flat, src)
    out_sorted = _moe(meta, x_sorted, w1, w2, c_proj)
    routed_flat = _make_row_gather(S)(out_sorted, pos)
    final_flat = routed_flat + shared_flat

    return final_flat.reshape(b, s, d), logits2d.reshape(b, s, E)


# speedup vs baseline: 1.0665x; 1.0665x over previous
import functools

import jax
import jax.numpy as jnp
from jax import lax
from jax.experimental import pallas as pl
from jax.experimental.pallas import tpu as pltpu
from jax.experimental.pallas import tpu_sc as plsc

D = 768
H = 2048
E = 64
S = 2048
CHUNK = 128
SLOT_PAD = 2816
SH_CHUNKS = (S + E * 8 + CHUNK - 1) // CHUNK


def _router_body(x_ref, rw_ref, logits_ref, eid_ref):
    lg = jnp.dot(x_ref[...], rw_ref[...], preferred_element_type=jnp.float32)
    logits_ref[...] = lg
    maxv = jnp.max(lg, axis=1, keepdims=True)
    col = lax.broadcasted_iota(jnp.int32, lg.shape, 1)
    ids = jnp.min(jnp.where(lg == maxv, col, E), axis=1, keepdims=True)
    eid_ref[...] = jnp.broadcast_to(ids, lg.shape).astype(jnp.int32)


_router = pl.pallas_call(
    _router_body,
    out_shape=(
        jax.ShapeDtypeStruct((S, E), jnp.float32),
        jax.ShapeDtypeStruct((S, E), jnp.int32),
    ),
)


_NC = 2
_NSC = 16
_NW = _NC * _NSC


@functools.lru_cache(maxsize=None)
def _make_row_gather(n_rows):
    b_per_w = n_rows // _NW
    mesh = plsc.VectorSubcoreMesh(
        core_axis_name="c", subcore_axis_name="s",
        num_cores=_NC, num_subcores=_NSC)

    @functools.partial(
        pl.kernel,
        mesh=mesh,
        out_type=jax.ShapeDtypeStruct((n_rows, D), jnp.float32),
        scratch_types=[
            pltpu.VMEM((b_per_w,), jnp.int32),
            pltpu.VMEM((b_per_w, D), jnp.float32),
            pltpu.SemaphoreType.DMA,
        ],
    )
    def k(table_hbm, idx_hbm, out_hbm, idx_v, rows_v, sem):
        wid = lax.axis_index("s") * _NC + lax.axis_index("c")
        base = wid * b_per_w
        pltpu.sync_copy(idx_hbm.at[pl.ds(base, b_per_w)], idx_v)
        pltpu.async_copy(table_hbm.at[idx_v], rows_v, sem).wait()
        pltpu.sync_copy(rows_v, out_hbm.at[pl.ds(base, b_per_w)])

    return k


def _silu(v):
    return v * jax.nn.sigmoid(v)


def _moe_body(meta_ref, x_ref, w1_ref, w2_ref, cp_ref, out_ref):
    g = pl.program_id(0)
    start = pl.multiple_of(meta_ref[g], 8)
    count = meta_ref[E + 1 + g]
    nch = (count + CHUNK - 1) // CHUNK

    def body(i, carry):
        r0 = start + i * CHUNK
        rows = x_ref[pl.ds(r0, CHUNK), :]
        a = jnp.dot(rows, w1_ref[0], preferred_element_type=jnp.float32)
        b = jnp.dot(rows, w2_ref[0], preferred_element_type=jnp.float32)
        hh = _silu(a) * b
        out_ref[pl.ds(r0, CHUNK), :] = jnp.dot(
            hh, cp_ref[0], preferred_element_type=jnp.float32)
        return carry

    lax.fori_loop(0, nch, body, 0)


_moe_grid = pltpu.PrefetchScalarGridSpec(
    num_scalar_prefetch=1,
    grid=(E,),
    in_specs=[
        pl.BlockSpec((SLOT_PAD, D), lambda g, offs: (0, 0)),
        pl.BlockSpec((1, D, H), lambda g, offs: (g, 0, 0)),
        pl.BlockSpec((1, D, H), lambda g, offs: (g, 0, 0)),
        pl.BlockSpec((1, H, D), lambda g, offs: (g, 0, 0)),
    ],
    out_specs=pl.BlockSpec((SLOT_PAD, D), lambda g, offs: (0, 0)),
)

_moe = pl.pallas_call(
    _moe_body,
    grid_spec=_moe_grid,
    out_shape=jax.ShapeDtypeStruct((SLOT_PAD, D), jnp.float32),
    compiler_params=pltpu.CompilerParams(
        dimension_semantics=("arbitrary",),
    ),
)


def _shared_body(x_ref, sw1_ref, sw2_ref, sc_ref, out_ref):
    rows = x_ref[...]
    a = jnp.dot(rows, sw1_ref[...], preferred_element_type=jnp.float32)
    b = jnp.dot(rows, sw2_ref[...], preferred_element_type=jnp.float32)
    hh = _silu(a) * b
    out_ref[...] = jnp.dot(hh, sc_ref[...], preferred_element_type=jnp.float32)


_SH_BLK = 256
_shared = pl.pallas_call(
    _shared_body,
    grid=(S // _SH_BLK,),
    in_specs=[
        pl.BlockSpec((_SH_BLK, D), lambda g: (g, 0)),
        pl.BlockSpec((D, H), lambda g: (0, 0)),
        pl.BlockSpec((D, H), lambda g: (0, 0)),
        pl.BlockSpec((H, D), lambda g: (0, 0)),
    ],
    out_specs=pl.BlockSpec((_SH_BLK, D), lambda g: (g, 0)),
    out_shape=jax.ShapeDtypeStruct((S, D), jnp.float32),
)


def kernel(x, router_w, w1, w2, c_proj, s_w1, s_w2, s_c):
    b, s, d = x.shape
    x_flat = x.reshape(s, d)

    logits2d, eid2d = _router(x_flat, router_w)
    eid = eid2d[:, 0]

    perm = jnp.argsort(eid).astype(jnp.int32)
    counts = jnp.zeros((E,), jnp.int32).at[eid].add(1)
    offsets = jnp.concatenate(
        [jnp.zeros((1,), jnp.int32), jnp.cumsum(counts).astype(jnp.int32)])
    pc = (counts + 7) // 8 * 8
    aoff = jnp.concatenate(
        [jnp.zeros((1,), jnp.int32), jnp.cumsum(pc).astype(jnp.int32)])
    es = eid[perm]
    slot = aoff[es] + jnp.arange(S, dtype=jnp.int32) - offsets[es]
    src = jnp.zeros((SLOT_PAD,), jnp.int32).at[slot].set(perm)
    pos = jnp.zeros((S,), jnp.int32).at[perm].set(slot)
    meta = jnp.concatenate([aoff, counts])

    x_gate, _ = lax.optimization_barrier((x_flat, src))
    shared_flat = _shared(x_gate, s_w1[0], s_w2[0], s_c[0])
    x_sorted = _make_row_gather(SLOT_PAD)(x_flat, src)
    meta2, _ = lax.optimization_barrier((meta, shared_flat))
    out_sorted = _moe(meta2, x_sorted, w1, w2, c_proj)
    routed_flat = _make_row_gather(S)(out_sorted, pos)
    final_flat = routed_flat + shared_flat

    return final_flat.reshape(b, s, d), logits2d.reshape(b, s, E)


# speedup vs baseline: 1.1160x; 1.0464x over previous
import functools

import jax
import jax.numpy as jnp
from jax import lax
from jax.experimental import pallas as pl
from jax.experimental.pallas import tpu as pltpu
from jax.experimental.pallas import tpu_sc as plsc

D = 768
H = 2048
E = 64
S = 2048
CHUNK = 128
SLOT_PAD = 2816
SH_CHUNKS = (S + E * 8 + CHUNK - 1) // CHUNK


def _router_body(x_ref, rw_ref, logits_ref, eid_ref):
    lg = jnp.dot(x_ref[...], rw_ref[...], preferred_element_type=jnp.float32)
    logits_ref[...] = lg
    maxv = jnp.max(lg, axis=1, keepdims=True)
    col = lax.broadcasted_iota(jnp.int32, lg.shape, 1)
    ids = jnp.min(jnp.where(lg == maxv, col, E), axis=1, keepdims=True)
    eid_ref[...] = jnp.broadcast_to(ids, lg.shape).astype(jnp.int32)


_router = pl.pallas_call(
    _router_body,
    out_shape=(
        jax.ShapeDtypeStruct((S, E), jnp.float32),
        jax.ShapeDtypeStruct((S, E), jnp.int32),
    ),
)


_NC = 2
_NSC = 16
_NW = _NC * _NSC


@functools.lru_cache(maxsize=None)
def _make_row_gather(n_rows):
    b_per_w = n_rows // _NW
    mesh = plsc.VectorSubcoreMesh(
        core_axis_name="c", subcore_axis_name="s",
        num_cores=_NC, num_subcores=_NSC)

    @functools.partial(
        pl.kernel,
        mesh=mesh,
        out_type=jax.ShapeDtypeStruct((n_rows, D), jnp.float32),
        scratch_types=[
            pltpu.VMEM((b_per_w,), jnp.int32),
            pltpu.VMEM((b_per_w, D), jnp.float32),
            pltpu.SemaphoreType.DMA,
        ],
    )
    def k(table_hbm, idx_hbm, out_hbm, idx_v, rows_v, sem):
        wid = lax.axis_index("s") * _NC + lax.axis_index("c")
        base = wid * b_per_w
        pltpu.sync_copy(idx_hbm.at[pl.ds(base, b_per_w)], idx_v)
        pltpu.async_copy(table_hbm.at[idx_v], rows_v, sem).wait()
        pltpu.sync_copy(rows_v, out_hbm.at[pl.ds(base, b_per_w)])

    return k


@functools.lru_cache(maxsize=None)
def _make_row_scatter():
    b_per_w = S // _NW
    mesh = plsc.VectorSubcoreMesh(
        core_axis_name="c", subcore_axis_name="s",
        num_cores=_NC, num_subcores=_NSC)

    @functools.partial(
        pl.kernel,
        mesh=mesh,
        out_type=jax.ShapeDtypeStruct((SLOT_PAD, D), jnp.float32),
        scratch_types=[
            pltpu.VMEM((b_per_w,), jnp.int32),
            pltpu.VMEM((b_per_w, D), jnp.float32),
            pltpu.SemaphoreType.DMA,
        ],
    )
    def k(x_hbm, idx_hbm, out_hbm, idx_v, rows_v, sem):
        wid = lax.axis_index("s") * _NC + lax.axis_index("c")
        base = wid * b_per_w
        pltpu.sync_copy(idx_hbm.at[pl.ds(base, b_per_w)], idx_v)
        pltpu.sync_copy(x_hbm.at[pl.ds(base, b_per_w)], rows_v)
        pltpu.async_copy(rows_v, out_hbm.at[idx_v], sem).wait()

    return k


def _silu(v):
    return v * jax.nn.sigmoid(v)


def _moe_body(meta_ref, x_ref, w1_ref, w2_ref, cp_ref, out_ref):
    g = pl.program_id(0)
    start = pl.multiple_of(meta_ref[g], 8)
    count = meta_ref[E + 1 + g]
    nch = (count + CHUNK - 1) // CHUNK

    def body(i, carry):
        r0 = start + i * CHUNK
        rows = x_ref[pl.ds(r0, CHUNK), :]
        a = jnp.dot(rows, w1_ref[0], preferred_element_type=jnp.float32)
        b = jnp.dot(rows, w2_ref[0], preferred_element_type=jnp.float32)
        hh = _silu(a) * b
        out_ref[pl.ds(r0, CHUNK), :] = jnp.dot(
            hh, cp_ref[0], preferred_element_type=jnp.float32)
        return carry

    lax.fori_loop(0, nch, body, 0)


_moe_grid = pltpu.PrefetchScalarGridSpec(
    num_scalar_prefetch=1,
    grid=(E,),
    in_specs=[
        pl.BlockSpec((SLOT_PAD, D), lambda g, offs: (0, 0)),
        pl.BlockSpec((1, D, H), lambda g, offs: (g, 0, 0)),
        pl.BlockSpec((1, D, H), lambda g, offs: (g, 0, 0)),
        pl.BlockSpec((1, H, D), lambda g, offs: (g, 0, 0)),
    ],
    out_specs=pl.BlockSpec((SLOT_PAD, D), lambda g, offs: (0, 0)),
)

_moe = pl.pallas_call(
    _moe_body,
    grid_spec=_moe_grid,
    out_shape=jax.ShapeDtypeStruct((SLOT_PAD, D), jnp.float32),
    compiler_params=pltpu.CompilerParams(
        dimension_semantics=("arbitrary",),
    ),
)


def _shared_body(x_ref, sw1_ref, sw2_ref, sc_ref, out_ref):
    rows = x_ref[...]
    a = jnp.dot(rows, sw1_ref[...], preferred_element_type=jnp.float32)
    b = jnp.dot(rows, sw2_ref[...], preferred_element_type=jnp.float32)
    hh = _silu(a) * b
    out_ref[...] = jnp.dot(hh, sc_ref[...], preferred_element_type=jnp.float32)


_SH_BLK = 256
_shared = pl.pallas_call(
    _shared_body,
    grid=(S // _SH_BLK,),
    in_specs=[
        pl.BlockSpec((_SH_BLK, D), lambda g: (g, 0)),
        pl.BlockSpec((D, H), lambda g: (0, 0)),
        pl.BlockSpec((D, H), lambda g: (0, 0)),
        pl.BlockSpec((H, D), lambda g: (0, 0)),
    ],
    out_specs=pl.BlockSpec((_SH_BLK, D), lambda g: (g, 0)),
    out_shape=jax.ShapeDtypeStruct((S, D), jnp.float32),
)


def kernel(x, router_w, w1, w2, c_proj, s_w1, s_w2, s_c):
    b, s, d = x.shape
    x_flat = x.reshape(s, d)

    logits2d, eid2d = _router(x_flat, router_w)
    eid = eid2d[:, 0]

    perm = jnp.argsort(eid).astype(jnp.int32)
    counts = jnp.zeros((E,), jnp.int32).at[eid].add(1)
    offsets = jnp.concatenate(
        [jnp.zeros((1,), jnp.int32), jnp.cumsum(counts).astype(jnp.int32)])
    pc = (counts + 7) // 8 * 8
    aoff = jnp.concatenate(
        [jnp.zeros((1,), jnp.int32), jnp.cumsum(pc).astype(jnp.int32)])
    es = eid[perm]
    slot = aoff[es] + jnp.arange(S, dtype=jnp.int32) - offsets[es]
    pos = jnp.zeros((S,), jnp.int32).at[perm].set(slot)
    meta = jnp.concatenate([aoff, counts])

    x_gate, _ = lax.optimization_barrier((x_flat, pos))
    shared_flat = _shared(x_gate, s_w1[0], s_w2[0], s_c[0])
    x_sorted = _make_row_scatter()(x_flat, pos)
    meta2, _ = lax.optimization_barrier((meta, shared_flat))
    out_sorted = _moe(meta2, x_sorted, w1, w2, c_proj)
    routed_flat = _make_row_gather(S)(out_sorted, pos)
    final_flat = routed_flat + shared_flat

    return final_flat.reshape(b, s, d), logits2d.reshape(b, s, E)


# speedup vs baseline: 1.2502x; 1.1202x over previous
import functools

import jax
import jax.numpy as jnp
from jax import lax
from jax.experimental import pallas as pl
from jax.experimental.pallas import tpu as pltpu
from jax.experimental.pallas import tpu_sc as plsc

D = 768
H = 2048
E = 64
S = 2048
CHUNK = 128
SLOT_PAD = 2816
SH_CHUNKS = (S + E * 8 + CHUNK - 1) // CHUNK


def _router_body(x_ref, rw_ref, logits_ref, pos_ref, aoff_ref, cnt_ref):
    lg = jnp.dot(x_ref[...], rw_ref[...], preferred_element_type=jnp.float32)
    logits_ref[...] = lg
    maxv = jnp.max(lg, axis=1, keepdims=True)
    col = lax.broadcasted_iota(jnp.int32, lg.shape, 1)
    ids = jnp.min(jnp.where(lg == maxv, col, E), axis=1, keepdims=True)

    onehot = (col == ids).astype(jnp.float32)
    counts = jnp.sum(onehot, axis=0, keepdims=True)
    pc = jnp.floor((counts + 7.0) * 0.125) * 8.0
    pc128 = jnp.concatenate(
        [pc, jnp.zeros((1, 128 - E), jnp.float32)], axis=1)
    r128 = lax.broadcasted_iota(jnp.int32, (128, 128), 0)
    c128 = lax.broadcasted_iota(jnp.int32, (128, 128), 1)
    tri128 = (r128 < c128).astype(jnp.float32)
    aoff = jnp.dot(pc128, tri128, preferred_element_type=jnp.float32)
    rt = lax.broadcasted_iota(jnp.int32, (S, S), 0)
    ct = lax.broadcasted_iota(jnp.int32, (S, S), 1)
    ltri = (ct < rt).astype(jnp.float32)
    run = jnp.dot(ltri, onehot, preferred_element_type=jnp.float32)
    rank = jnp.sum(run * onehot, axis=1, keepdims=True)
    aofft = jnp.sum(onehot * aoff[:, :E], axis=1, keepdims=True)
    slot = (aofft + rank).astype(jnp.int32)

    pos_ref[...] = jnp.broadcast_to(slot, (S, E))
    aoff_ref[...] = jnp.broadcast_to(aoff.astype(jnp.int32), (8, 128))
    counts128 = jnp.concatenate(
        [counts, jnp.zeros((1, 128 - E), jnp.float32)], axis=1)
    cnt_ref[...] = jnp.broadcast_to(counts128.astype(jnp.int32), (8, 128))


_router = pl.pallas_call(
    _router_body,
    out_shape=(
        jax.ShapeDtypeStruct((S, E), jnp.float32),
        jax.ShapeDtypeStruct((S, E), jnp.int32),
        jax.ShapeDtypeStruct((8, 128), jnp.int32),
        jax.ShapeDtypeStruct((8, 128), jnp.int32),
    ),
)


_NC = 2
_NSC = 16
_NW = _NC * _NSC


@functools.lru_cache(maxsize=None)
def _make_row_gather(n_rows):
    b_per_w = n_rows // _NW
    mesh = plsc.VectorSubcoreMesh(
        core_axis_name="c", subcore_axis_name="s",
        num_cores=_NC, num_subcores=_NSC)

    @functools.partial(
        pl.kernel,
        mesh=mesh,
        out_type=jax.ShapeDtypeStruct((n_rows, D), jnp.float32),
        scratch_types=[
            pltpu.VMEM((b_per_w,), jnp.int32),
            pltpu.VMEM((b_per_w, D), jnp.float32),
            pltpu.SemaphoreType.DMA,
        ],
    )
    def k(table_hbm, idx_hbm, out_hbm, idx_v, rows_v, sem):
        wid = lax.axis_index("s") * _NC + lax.axis_index("c")
        base = wid * b_per_w
        pltpu.sync_copy(idx_hbm.at[pl.ds(base, b_per_w)], idx_v)
        pltpu.async_copy(table_hbm.at[idx_v], rows_v, sem).wait()
        pltpu.sync_copy(rows_v, out_hbm.at[pl.ds(base, b_per_w)])

    return k


@functools.lru_cache(maxsize=None)
def _make_row_scatter():
    b_per_w = S // _NW
    mesh = plsc.VectorSubcoreMesh(
        core_axis_name="c", subcore_axis_name="s",
        num_cores=_NC, num_subcores=_NSC)

    @functools.partial(
        pl.kernel,
        mesh=mesh,
        out_type=jax.ShapeDtypeStruct((SLOT_PAD, D), jnp.float32),
        scratch_types=[
            pltpu.VMEM((b_per_w,), jnp.int32),
            pltpu.VMEM((b_per_w, D), jnp.float32),
            pltpu.SemaphoreType.DMA,
        ],
    )
    def k(x_hbm, idx_hbm, out_hbm, idx_v, rows_v, sem):
        wid = lax.axis_index("s") * _NC + lax.axis_index("c")
        base = wid * b_per_w
        pltpu.sync_copy(idx_hbm.at[pl.ds(base, b_per_w)], idx_v)
        pltpu.sync_copy(x_hbm.at[pl.ds(base, b_per_w)], rows_v)
        pltpu.async_copy(rows_v, out_hbm.at[idx_v], sem).wait()

    return k


def _silu(v):
    return v * jax.nn.sigmoid(v)


def _moe_body(meta_ref, x_ref, w1_ref, w2_ref, cp_ref, out_ref):
    g = pl.program_id(0)
    start = pl.multiple_of(meta_ref[g], 8)
    count = meta_ref[E + 1 + g]
    nch = (count + CHUNK - 1) // CHUNK

    def body(i, carry):
        r0 = start + i * CHUNK
        rows = x_ref[pl.ds(r0, CHUNK), :]
        a = jnp.dot(rows, w1_ref[0], preferred_element_type=jnp.float32)
        b = jnp.dot(rows, w2_ref[0], preferred_element_type=jnp.float32)
        hh = _silu(a) * b
        out_ref[pl.ds(r0, CHUNK), :] = jnp.dot(
            hh, cp_ref[0], preferred_element_type=jnp.float32)
        return carry

    lax.fori_loop(0, nch, body, 0)


_moe_grid = pltpu.PrefetchScalarGridSpec(
    num_scalar_prefetch=1,
    grid=(E,),
    in_specs=[
        pl.BlockSpec((SLOT_PAD, D), lambda g, offs: (0, 0)),
        pl.BlockSpec((1, D, H), lambda g, offs: (g, 0, 0)),
        pl.BlockSpec((1, D, H), lambda g, offs: (g, 0, 0)),
        pl.BlockSpec((1, H, D), lambda g, offs: (g, 0, 0)),
    ],
    out_specs=pl.BlockSpec((SLOT_PAD, D), lambda g, offs: (0, 0)),
)

_moe = pl.pallas_call(
    _moe_body,
    grid_spec=_moe_grid,
    out_shape=jax.ShapeDtypeStruct((SLOT_PAD, D), jnp.float32),
    compiler_params=pltpu.CompilerParams(
        dimension_semantics=("arbitrary",),
    ),
)


def _shared_body(x_ref, sw1_ref, sw2_ref, sc_ref, out_ref):
    rows = x_ref[...]
    a = jnp.dot(rows, sw1_ref[...], preferred_element_type=jnp.float32)
    b = jnp.dot(rows, sw2_ref[...], preferred_element_type=jnp.float32)
    hh = _silu(a) * b
    out_ref[...] = jnp.dot(hh, sc_ref[...], preferred_element_type=jnp.float32)


_SH_BLK = 256
_shared = pl.pallas_call(
    _shared_body,
    grid=(S // _SH_BLK,),
    in_specs=[
        pl.BlockSpec((_SH_BLK, D), lambda g: (g, 0)),
        pl.BlockSpec((D, H), lambda g: (0, 0)),
        pl.BlockSpec((D, H), lambda g: (0, 0)),
        pl.BlockSpec((H, D), lambda g: (0, 0)),
    ],
    out_specs=pl.BlockSpec((_SH_BLK, D), lambda g: (g, 0)),
    out_shape=jax.ShapeDtypeStruct((S, D), jnp.float32),
)


def kernel(x, router_w, w1, w2, c_proj, s_w1, s_w2, s_c):
    b, s, d = x.shape
    x_flat = x.reshape(s, d)

    logits2d, pos2d, aoff2d, cnt2d = _router(x_flat, router_w)
    pos = pos2d[:, 0]
    meta = jnp.concatenate([aoff2d[0, :E + 1], cnt2d[0, :E]])

    x_gate, _ = lax.optimization_barrier((x_flat, pos))
    shared_flat = _shared(x_gate, s_w1[0], s_w2[0], s_c[0])
    x_sorted = _make_row_scatter()(x_flat, pos)
    meta2, _ = lax.optimization_barrier((meta, shared_flat))
    out_sorted = _moe(meta2, x_sorted, w1, w2, c_proj)
    routed_flat = _make_row_gather(S)(out_sorted, pos)
    final_flat = routed_flat + shared_flat

    return final_flat.reshape(b, s, d), logits2d.reshape(b, s, E)


# speedup vs baseline: 1.2511x; 1.0007x over previous
import functools

import jax
import jax.numpy as jnp
from jax import lax
from jax.experimental import pallas as pl
from jax.experimental.pallas import tpu as pltpu
from jax.experimental.pallas import tpu_sc as plsc

D = 768
H = 2048
E = 64
S = 2048
CHUNK = 128
SLOT_PAD = 2816
SH_CHUNKS = (S + E * 8 + CHUNK - 1) // CHUNK


def _router_body(x_ref, rw_ref, logits_ref, pos_ref, aoff_ref, cnt_ref):
    lg = jnp.dot(x_ref[...], rw_ref[...], preferred_element_type=jnp.float32)
    logits_ref[...] = lg
    maxv = jnp.max(lg, axis=1, keepdims=True)
    col = lax.broadcasted_iota(jnp.int32, lg.shape, 1)
    ids = jnp.min(jnp.where(lg == maxv, col, E), axis=1, keepdims=True)

    onehot = (col == ids).astype(jnp.float32)
    counts = jnp.sum(onehot, axis=0, keepdims=True)
    pc = jnp.floor((counts + 7.0) * 0.125) * 8.0
    pc128 = jnp.concatenate(
        [pc, jnp.zeros((1, 128 - E), jnp.float32)], axis=1)
    r128 = lax.broadcasted_iota(jnp.int32, (128, 128), 0)
    c128 = lax.broadcasted_iota(jnp.int32, (128, 128), 1)
    tri128 = (r128 < c128).astype(jnp.float32)
    aoff = jnp.dot(pc128, tri128, preferred_element_type=jnp.float32)
    rt = lax.broadcasted_iota(jnp.int32, (S, S), 0)
    ct = lax.broadcasted_iota(jnp.int32, (S, S), 1)
    ltri = (ct < rt).astype(jnp.float32)
    run = jnp.dot(ltri, onehot, preferred_element_type=jnp.float32)
    rank = jnp.sum(run * onehot, axis=1, keepdims=True)
    aofft = jnp.sum(onehot * aoff[:, :E], axis=1, keepdims=True)
    slot = (aofft + rank).astype(jnp.int32)

    pos_ref[...] = jnp.broadcast_to(slot, (S, E))
    aoff_ref[...] = jnp.broadcast_to(aoff.astype(jnp.int32), (8, 128))
    counts128 = jnp.concatenate(
        [counts, jnp.zeros((1, 128 - E), jnp.float32)], axis=1)
    cnt_ref[...] = jnp.broadcast_to(counts128.astype(jnp.int32), (8, 128))


_router = pl.pallas_call(
    _router_body,
    out_shape=(
        jax.ShapeDtypeStruct((S, E), jnp.float32),
        jax.ShapeDtypeStruct((S, E), jnp.int32),
        jax.ShapeDtypeStruct((8, 128), jnp.int32),
        jax.ShapeDtypeStruct((8, 128), jnp.int32),
    ),
)


_NC = 2
_NSC = 16
_NW = _NC * _NSC


@functools.lru_cache(maxsize=None)
def _make_row_gather(n_rows):
    b_per_w = n_rows // _NW
    mesh = plsc.VectorSubcoreMesh(
        core_axis_name="c", subcore_axis_name="s",
        num_cores=_NC, num_subcores=_NSC)

    @functools.partial(
        pl.kernel,
        mesh=mesh,
        out_type=jax.ShapeDtypeStruct((n_rows, D), jnp.float32),
        scratch_types=[
            pltpu.VMEM((b_per_w,), jnp.int32),
            pltpu.VMEM((b_per_w, D), jnp.float32),
            pltpu.SemaphoreType.DMA,
        ],
    )
    def k(table_hbm, idx_hbm, out_hbm, idx_v, rows_v, sem):
        wid = lax.axis_index("s") * _NC + lax.axis_index("c")
        base = wid * b_per_w
        pltpu.sync_copy(idx_hbm.at[pl.ds(base, b_per_w)], idx_v)
        pltpu.async_copy(table_hbm.at[idx_v], rows_v, sem).wait()
        pltpu.sync_copy(rows_v, out_hbm.at[pl.ds(base, b_per_w)])

    return k


@functools.lru_cache(maxsize=None)
def _make_sc_warmup():
    mesh = plsc.VectorSubcoreMesh(
        core_axis_name="c", subcore_axis_name="s",
        num_cores=_NC, num_subcores=_NSC)

    @functools.partial(
        pl.kernel,
        mesh=mesh,
        out_type=jax.ShapeDtypeStruct((_NW * 8, D), jnp.float32),
        scratch_types=[pltpu.VMEM((8, D), jnp.float32)],
    )
    def k(x_hbm, out_hbm, rows_v):
        wid = lax.axis_index("s") * _NC + lax.axis_index("c")
        base = wid * 8
        pltpu.sync_copy(x_hbm.at[pl.ds(base, 8)], rows_v)
        pltpu.sync_copy(rows_v, out_hbm.at[pl.ds(base, 8)])

    return k


@functools.lru_cache(maxsize=None)
def _make_row_scatter():
    b_per_w = S // _NW
    mesh = plsc.VectorSubcoreMesh(
        core_axis_name="c", subcore_axis_name="s",
        num_cores=_NC, num_subcores=_NSC)

    @functools.partial(
        pl.kernel,
        mesh=mesh,
        out_type=jax.ShapeDtypeStruct((SLOT_PAD, D), jnp.float32),
        scratch_types=[
            pltpu.VMEM((b_per_w,), jnp.int32),
            pltpu.VMEM((b_per_w, D), jnp.float32),
            pltpu.SemaphoreType.DMA,
        ],
    )
    def k(x_hbm, idx_hbm, out_hbm, idx_v, rows_v, sem):
        wid = lax.axis_index("s") * _NC + lax.axis_index("c")
        base = wid * b_per_w
        pltpu.sync_copy(idx_hbm.at[pl.ds(base, b_per_w)], idx_v)
        pltpu.sync_copy(x_hbm.at[pl.ds(base, b_per_w)], rows_v)
        pltpu.async_copy(rows_v, out_hbm.at[idx_v], sem).wait()

    return k


def _silu(v):
    return v * jax.nn.sigmoid(v)


def _moe_body(meta_ref, x_ref, w1_ref, w2_ref, cp_ref, out_ref):
    g = pl.program_id(0)
    start = pl.multiple_of(meta_ref[g], 8)
    count = meta_ref[E + 1 + g]
    nch = (count + CHUNK - 1) // CHUNK

    def body(i, carry):
        r0 = start + i * CHUNK
        rows = x_ref[pl.ds(r0, CHUNK), :]
        a = jnp.dot(rows, w1_ref[0], preferred_element_type=jnp.float32)
        b = jnp.dot(rows, w2_ref[0], preferred_element_type=jnp.float32)
        hh = _silu(a) * b
        out_ref[pl.ds(r0, CHUNK), :] = jnp.dot(
            hh, cp_ref[0], preferred_element_type=jnp.float32)
        return carry

    lax.fori_loop(0, nch, body, 0)


_moe_grid = pltpu.PrefetchScalarGridSpec(
    num_scalar_prefetch=1,
    grid=(E,),
    in_specs=[
        pl.BlockSpec((SLOT_PAD, D), lambda g, offs: (0, 0)),
        pl.BlockSpec((1, D, H), lambda g, offs: (g, 0, 0)),
        pl.BlockSpec((1, D, H), lambda g, offs: (g, 0, 0)),
        pl.BlockSpec((1, H, D), lambda g, offs: (g, 0, 0)),
    ],
    out_specs=pl.BlockSpec((SLOT_PAD, D), lambda g, offs: (0, 0)),
)

_moe = pl.pallas_call(
    _moe_body,
    grid_spec=_moe_grid,
    out_shape=jax.ShapeDtypeStruct((SLOT_PAD, D), jnp.float32),
    compiler_params=pltpu.CompilerParams(
        dimension_semantics=("arbitrary",),
    ),
)


def _shared_body(x_ref, sw1_ref, sw2_ref, sc_ref, out_ref):
    rows = x_ref[...]
    a = jnp.dot(rows, sw1_ref[...], preferred_element_type=jnp.float32)
    b = jnp.dot(rows, sw2_ref[...], preferred_element_type=jnp.float32)
    hh = _silu(a) * b
    out_ref[...] = jnp.dot(hh, sc_ref[...], preferred_element_type=jnp.float32)


_SH_BLK = 256
_shared = pl.pallas_call(
    _shared_body,
    grid=(S // _SH_BLK,),
    in_specs=[
        pl.BlockSpec((_SH_BLK, D), lambda g: (g, 0)),
        pl.BlockSpec((D, H), lambda g: (0, 0)),
        pl.BlockSpec((D, H), lambda g: (0, 0)),
        pl.BlockSpec((H, D), lambda g: (0, 0)),
    ],
    out_specs=pl.BlockSpec((_SH_BLK, D), lambda g: (g, 0)),
    out_shape=jax.ShapeDtypeStruct((S, D), jnp.float32),
)


def kernel(x, router_w, w1, w2, c_proj, s_w1, s_w2, s_c):
    b, s, d = x.shape
    x_flat = x.reshape(s, d)

    logits2d, pos2d, aoff2d, cnt2d = _router(x_flat, router_w)
    pos = pos2d[:, 0]
    meta = jnp.concatenate([aoff2d[0, :E + 1], cnt2d[0, :E]])

    x_gate, _ = lax.optimization_barrier((x_flat, pos))
    shared_flat = _shared(x_gate, s_w1[0], s_w2[0], s_c[0])
    x_sorted = _make_row_scatter()(x_flat, pos)
    warm = _make_sc_warmup()(x_flat)
    meta2, _ = lax.optimization_barrier((meta, shared_flat))
    out_sorted = _moe(meta2, x_sorted, w1, w2, c_proj)
    out_gate, _ = lax.optimization_barrier((out_sorted, warm))
    routed_flat = _make_row_gather(S)(out_gate, pos)
    final_flat = routed_flat + shared_flat

    return final_flat.reshape(b, s, d), logits2d.reshape(b, s, E)
